# Initial kernel scaffold; baseline (speedup 1.0000x reference)
import jax
import jax.numpy as jnp
from jax.experimental import pallas as pl


def kernel(token_ids, token_embed):
    raise NotImplementedError("write your pallas kernel here")


# speedup vs baseline: 9.2217x; 9.2217x over previous
import functools
import math

import jax
import jax.numpy as jnp
from jax import lax
from jax.experimental import pallas as pl
from jax.experimental.pallas import tpu as pltpu
from jax.experimental.pallas import tpu_sc as plsc

LANES = 16
CHUNK = 128
NBUF = 2


def kernel(token_ids, token_embed):
    B, L = token_ids.shape
    V, D = token_embed.shape
    scale = math.sqrt(D)
    N = B * L

    info = plsc.get_sparse_core_info()
    NC, NS = info.num_cores, info.num_subcores
    NW = NC * NS
    assert N % (NW * CHUNK) == 0
    per_w = N // NW
    n_chunks = per_w // CHUNK
    assert n_chunks % NBUF == 0
    R = n_chunks // NBUF

    idx = token_ids.reshape(NW, n_chunks, CHUNK).astype(jnp.int32)
    mesh = plsc.VectorSubcoreMesh(core_axis_name="c", subcore_axis_name="s")

    @functools.partial(
        pl.kernel,
        mesh=mesh,
        out_type=jax.ShapeDtypeStruct((N, D), jnp.float32),
        scratch_types=[
            pltpu.VMEM((n_chunks, CHUNK), jnp.int32),
            pltpu.VMEM((NBUF, CHUNK, D), jnp.float32),
            pltpu.VMEM((NBUF, CHUNK, D), jnp.float32),
            pltpu.SemaphoreType.DMA,
            pltpu.SemaphoreType.DMA,
            pltpu.SemaphoreType.DMA,
            pltpu.SemaphoreType.DMA,
        ],
    )
    def sc_kernel(idx_hbm, tab_hbm, out_hbm, idx_v, gbuf, sbuf, g0, g1, s0, s1):
        gsem = [g0, g1]
        ssem = [s0, s1]
        wid = lax.axis_index("s") * NC + lax.axis_index("c")
        row0 = wid * per_w
        pltpu.sync_copy(idx_hbm.at[wid], idx_v)

        def g_start(b, ch):
            pltpu.async_copy(tab_hbm.at[idx_v.at[ch]], gbuf.at[b], gsem[b])

        def g_wait(b):
            pltpu.make_async_copy(
                tab_hbm.at[pl.ds(0, CHUNK)], gbuf.at[b], gsem[b]).wait()

        def s_start(b, ch):
            pltpu.async_copy(
                sbuf.at[b], out_hbm.at[pl.ds(row0 + ch * CHUNK, CHUNK)], ssem[b])

        def s_wait(b):
            pltpu.make_async_copy(
                sbuf.at[b], out_hbm.at[pl.ds(0, CHUNK)], ssem[b]).wait()

        for b in range(NBUF):
            g_start(b, b)

        def round_body(j, carry):
            for b in range(NBUF):
                ch = j * NBUF + b
                g_wait(b)

                @pl.when(j > 0)
                def _():
                    s_wait(b)

                def row_body(r, c):
                    for g in range(D // LANES):
                        sl = pl.ds(g * LANES, LANES)
                        sbuf[b, r, sl] = gbuf[b, r, sl] * scale
                    return c

                lax.fori_loop(0, CHUNK, row_body, 0)
                s_start(b, ch)

                @pl.when(j < R - 1)
                def _():
                    g_start(b, ch + NBUF)
            return carry

        lax.fori_loop(0, R, round_body, 0)
        for b in range(NBUF):
            s_wait(b)

    out = sc_kernel(idx, token_embed)
    return out.reshape(B, L, D)
